# baseline (speedup 1.0000x reference)
import functools
import jax
import jax.numpy as jnp
import numpy as np
from jax.experimental import pallas as pl
from jax.experimental.pallas import tpu as pltpu

N = 50000
E = 800000
HID = 64
HEADS = 4
CH = 16
NG = 1024
BN = 1000
BE = 8000


def _ln(v, g, b, eps=1e-5):
    mu = jnp.mean(v, axis=-1, keepdims=True)
    var = jnp.mean((v - mu) ** 2, axis=-1, keepdims=True)
    return (v - mu) * jax.lax.rsqrt(var + eps) * g + b


def _full(shape):
    return pl.BlockSpec(shape, lambda i: (0, 0))


def _atom_body(xf_ref, D_ref, base_ref, W_ref, b_ref, g_ref, be_ref,
               pos0_ref, posd_ref, o_ref):
    h0 = jnp.dot(xf_ref[...], D_ref[...], preferred_element_type=jnp.float32) + base_ref[...]
    h1 = jnp.dot(h0, W_ref[...], preferred_element_type=jnp.float32) + b_ref[...]
    h1 = jnp.maximum(_ln(h1, g_ref[...], be_ref[...]), 0.0)
    x2 = xf_ref[:, 2:3]
    o_ref[...] = h1 + pos0_ref[...] + x2 * posd_ref[...]


def _atom_encode(xf, D, base, W, b, g, be, pos0, posd):
    return pl.pallas_call(
        _atom_body,
        grid=(N // BN,),
        in_specs=[
            pl.BlockSpec((BN, 9), lambda i: (i, 0)),
            _full((9, HID)), _full((1, HID)), _full((HID, HID)),
            _full((1, HID)), _full((1, HID)), _full((1, HID)),
            _full((1, HID)), _full((1, HID)),
        ],
        out_specs=pl.BlockSpec((BN, HID), lambda i: (i, 0)),
        out_shape=jax.ShapeDtypeStruct((N, HID), jnp.float32),
    )(xf, D, base, W, b, g, be, pos0, posd)


def _bond_body(ef_ref, D_ref, base_ref, W_ref, b_ref, g_ref, be_ref, o_ref):
    h0 = jnp.dot(ef_ref[...], D_ref[...], preferred_element_type=jnp.float32) + base_ref[...]
    h1 = jnp.dot(h0, W_ref[...], preferred_element_type=jnp.float32) + b_ref[...]
    o_ref[...] = jnp.maximum(_ln(h1, g_ref[...], be_ref[...]), 0.0)


def _bond_encode(ef, D, base, W, b, g, be):
    return pl.pallas_call(
        _bond_body,
        grid=(E // BE,),
        in_specs=[
            pl.BlockSpec((BE, 3), lambda i: (i, 0)),
            _full((3, HID)), _full((1, HID)), _full((HID, HID)),
            _full((1, HID)), _full((1, HID)), _full((1, HID)),
        ],
        out_specs=pl.BlockSpec((BE, HID), lambda i: (i, 0)),
        out_shape=jax.ShapeDtypeStruct((E, HID), jnp.float32),
    )(ef, D, base, W, b, g, be)


def _qkvs_body(h_ref, Wq, bq, Wk, bk, Wv, bv, Ws, bs,
               q_ref, k_ref, v_ref, s_ref):
    h = h_ref[...]
    q_ref[...] = jnp.dot(h, Wq[...], preferred_element_type=jnp.float32) + bq[...]
    k_ref[...] = jnp.dot(h, Wk[...], preferred_element_type=jnp.float32) + bk[...]
    v_ref[...] = jnp.dot(h, Wv[...], preferred_element_type=jnp.float32) + bv[...]
    s_ref[...] = jnp.dot(h, Ws[...], preferred_element_type=jnp.float32) + bs[...]


def _qkvs(h, lp):
    outs = pl.pallas_call(
        _qkvs_body,
        grid=(N // BN,),
        in_specs=[pl.BlockSpec((BN, HID), lambda i: (i, 0))] +
                 [_full((HID, HID)), _full((1, HID))] * 4,
        out_specs=[pl.BlockSpec((BN, HID), lambda i: (i, 0))] * 4,
        out_shape=[jax.ShapeDtypeStruct((N, HID), jnp.float32)] * 4,
    )(h, lp["Wq"], lp["bq"].reshape(1, HID), lp["Wk"], lp["bk"].reshape(1, HID),
      lp["Wv"], lp["bv"].reshape(1, HID), lp["Ws"], lp["bs"].reshape(1, HID))
    return outs


def _et_body(ea_ref, We, o_ref):
    o_ref[...] = jnp.dot(ea_ref[...], We[...], preferred_element_type=jnp.float32)


def _edge_transform(ea, We):
    return pl.pallas_call(
        _et_body,
        grid=(E // BE,),
        in_specs=[pl.BlockSpec((BE, HID), lambda i: (i, 0)), _full((HID, HID))],
        out_specs=pl.BlockSpec((BE, HID), lambda i: (i, 0)),
        out_shape=jax.ShapeDtypeStruct((E, HID), jnp.float32),
    )(ea, We)


def _post_body(agg_ref, hs_ref, h_ref, g_ref, b_ref, o_ref):
    out = agg_ref[...] + hs_ref[...]
    out = jnp.maximum(_ln(out, g_ref[...], b_ref[...]), 0.0)
    o_ref[...] = out + h_ref[...]


def _post(agg, hs, h, lp):
    return pl.pallas_call(
        _post_body,
        grid=(N // BN,),
        in_specs=[pl.BlockSpec((BN, HID), lambda i: (i, 0))] * 3 +
                 [_full((1, HID)), _full((1, HID))],
        out_specs=pl.BlockSpec((BN, HID), lambda i: (i, 0)),
        out_shape=jax.ShapeDtypeStruct((N, HID), jnp.float32),
    )(agg, hs, h, lp["ln_g"].reshape(1, HID), lp["ln_b"].reshape(1, HID))


def _pool_body(h_ref, W1, b1, W2, b2, o_ref):
    t = jnp.tanh(jnp.dot(h_ref[...], W1[...], preferred_element_type=jnp.float32) + b1[...])
    lg = jnp.dot(t, W2[...], preferred_element_type=jnp.float32) + b2[...]
    o_ref[...] = jnp.broadcast_to(lg, (BN, 8))


def _pool_logits(h, W1, b1, W2, b2):
    out = pl.pallas_call(
        _pool_body,
        grid=(N // BN,),
        in_specs=[pl.BlockSpec((BN, HID), lambda i: (i, 0)),
                  _full((HID, HID)), _full((1, HID)),
                  _full((HID, 1)), _full((1, 1))],
        out_specs=pl.BlockSpec((BN, 8), lambda i: (i, 0)),
        out_shape=jax.ShapeDtypeStruct((N, 8), jnp.float32),
    )(h, W1, b1.reshape(1, HID), W2, b2.reshape(1, 1))
    return out[:, 0]


def _proj_body(g_ref, W1, b1, g1, be1, W2, b2, g2, be2, W3, b3, o_ref):
    g = g_ref[...]
    g = jnp.maximum(_ln(jnp.dot(g, W1[...], preferred_element_type=jnp.float32) + b1[...],
                        g1[...], be1[...]), 0.0)
    g = jnp.maximum(_ln(jnp.dot(g, W2[...], preferred_element_type=jnp.float32) + b2[...],
                        g2[...], be2[...]), 0.0)
    g = jnp.dot(g, W3[...], preferred_element_type=jnp.float32) + b3[...]
    nrm = jnp.sqrt(jnp.sum(g * g, axis=-1, keepdims=True))
    o_ref[...] = g / jnp.maximum(nrm, 1e-12)


def _proj_head(g, p):
    H2 = HID * 2
    return pl.pallas_call(
        _proj_body,
        grid=(1,),
        in_specs=[_full((NG, HID)),
                  _full((HID, H2)), _full((1, H2)), _full((1, H2)), _full((1, H2)),
                  _full((H2, HID)), _full((1, HID)), _full((1, HID)), _full((1, HID)),
                  _full((HID, 128)), _full((1, 128))],
        out_specs=_full((NG, 128)),
        out_shape=jax.ShapeDtypeStruct((NG, 128), jnp.float32),
    )(g, p["proj_W1"], p["proj_b1"].reshape(1, H2), p["proj_g1"].reshape(1, H2),
      p["proj_be1"].reshape(1, H2), p["proj_W2"], p["proj_b2"].reshape(1, HID),
      p["proj_g2"].reshape(1, HID), p["proj_be2"].reshape(1, HID),
      p["proj_W3"], p["proj_b3"].reshape(1, 128))


def _two_row_collapse(tables, fw):
    sig = jax.nn.sigmoid(fw)
    base = sum(sig[i] * tables[i][0] for i in range(len(tables)))
    D = jnp.stack([sig[i] * (tables[i][1] - tables[i][0]) for i in range(len(tables))], 0)
    return base.reshape(1, HID), D


def kernel(params, x, edge_index, edge_attr, batch):
    p = params
    xf = x.astype(jnp.float32)

    src, dst = edge_index[0], edge_index[1]
    perm = jnp.argsort(dst)
    dst_s = dst[perm]
    src_s = src[perm]
    ef = edge_attr.astype(jnp.float32)[perm]

    base_a, D_a = _two_row_collapse(p["atom_emb"], p["atom_fw"])
    base_b, D_b = _two_row_collapse(p["bond_emb"], p["bond_fw"])
    pos0 = p["pos"][0].reshape(1, HID)
    posd = (p["pos"][1] - p["pos"][0]).reshape(1, HID)

    h = _atom_encode(xf, D_a, base_a, p["atom_W"], p["atom_b"].reshape(1, HID),
                     p["atom_g"].reshape(1, HID), p["atom_be"].reshape(1, HID),
                     pos0, posd)
    ea = _bond_encode(ef, D_b, base_b, p["bond_W"], p["bond_b"].reshape(1, HID),
                      p["bond_g"].reshape(1, HID), p["bond_be"].reshape(1, HID))

    for lp in p["layers"]:
        q, k, v, hs = _qkvs(h, lp)
        e = _edge_transform(ea, lp["We"])

        qh = q.reshape(N, HEADS, CH)
        kh = k.reshape(N, HEADS, CH)
        vh = v.reshape(N, HEADS, CH)
        eh = e.reshape(E, HEADS, CH)
        ke = kh[src_s] + eh
        logit = jnp.sum(qh[dst_s] * ke, axis=-1) * 0.25
        m = jax.ops.segment_max(logit, dst_s, num_segments=N,
                                indices_are_sorted=True)
        ex = jnp.exp(logit - m[dst_s])
        s = jax.ops.segment_sum(ex, dst_s, num_segments=N,
                                indices_are_sorted=True)
        alpha = ex / (s[dst_s] + 1e-16)
        msg = (vh[src_s] + eh) * alpha[..., None]
        agg = jax.ops.segment_sum(msg, dst_s, num_segments=N,
                                  indices_are_sorted=True).reshape(N, HID)
        h = _post(agg, hs, h, lp)

    logits = _pool_logits(h, p["pool_W1"], p["pool_b1"], p["pool_W2"], p["pool_b2"])
    m = jax.ops.segment_max(logits, batch, num_segments=NG, indices_are_sorted=True)
    ex = jnp.exp(logits - m[batch])
    s = jax.ops.segment_sum(ex, batch, num_segments=NG, indices_are_sorted=True)
    w = ex / (s[batch] + 1e-16)
    g = jax.ops.segment_sum(h * w[:, None], batch, num_segments=NG,
                            indices_are_sorted=True)
    return _proj_head(g, p)


# speedup vs baseline: 1.1255x; 1.1255x over previous
import functools
import jax
import jax.numpy as jnp
import numpy as np
from jax.experimental import pallas as pl
from jax.experimental.pallas import tpu as pltpu

N = 50000
E = 800000
HID = 64
HEADS = 4
CH = 16
NG = 1024
BN = 1000
BE = 8000


def _ln(v, g, b, eps=1e-5):
    mu = jnp.mean(v, axis=-1, keepdims=True)
    var = jnp.mean((v - mu) ** 2, axis=-1, keepdims=True)
    return (v - mu) * jax.lax.rsqrt(var + eps) * g + b


def _full(shape):
    return pl.BlockSpec(shape, lambda i: (0, 0))


def _atom_body(xf_ref, D_ref, base_ref, W_ref, b_ref, g_ref, be_ref,
               pos0_ref, posd_ref, o_ref):
    h0 = jnp.dot(xf_ref[...], D_ref[...], preferred_element_type=jnp.float32) + base_ref[...]
    h1 = jnp.dot(h0, W_ref[...], preferred_element_type=jnp.float32) + b_ref[...]
    h1 = jnp.maximum(_ln(h1, g_ref[...], be_ref[...]), 0.0)
    x2 = xf_ref[:, 2:3]
    o_ref[...] = h1 + pos0_ref[...] + x2 * posd_ref[...]


def _atom_encode(xf, D, base, W, b, g, be, pos0, posd):
    return pl.pallas_call(
        _atom_body,
        grid=(N // BN,),
        in_specs=[
            pl.BlockSpec((BN, 9), lambda i: (i, 0)),
            _full((9, HID)), _full((1, HID)), _full((HID, HID)),
            _full((1, HID)), _full((1, HID)), _full((1, HID)),
            _full((1, HID)), _full((1, HID)),
        ],
        out_specs=pl.BlockSpec((BN, HID), lambda i: (i, 0)),
        out_shape=jax.ShapeDtypeStruct((N, HID), jnp.float32),
    )(xf, D, base, W, b, g, be, pos0, posd)


def _bond_body(ef_ref, D_ref, base_ref, W_ref, b_ref, g_ref, be_ref, o_ref):
    h0 = jnp.dot(ef_ref[...], D_ref[...], preferred_element_type=jnp.float32) + base_ref[...]
    h1 = jnp.dot(h0, W_ref[...], preferred_element_type=jnp.float32) + b_ref[...]
    o_ref[...] = jnp.maximum(_ln(h1, g_ref[...], be_ref[...]), 0.0)


def _bond_encode(ef, D, base, W, b, g, be):
    return pl.pallas_call(
        _bond_body,
        grid=(E // BE,),
        in_specs=[
            pl.BlockSpec((BE, 3), lambda i: (i, 0)),
            _full((3, HID)), _full((1, HID)), _full((HID, HID)),
            _full((1, HID)), _full((1, HID)), _full((1, HID)),
        ],
        out_specs=pl.BlockSpec((BE, HID), lambda i: (i, 0)),
        out_shape=jax.ShapeDtypeStruct((E, HID), jnp.float32),
    )(ef, D, base, W, b, g, be)


def _qkvs_body(h_ref, Wq, bq, Wk, bk, Wv, bv, Ws, bs,
               q_ref, k_ref, v_ref, s_ref):
    h = h_ref[...]
    q_ref[...] = jnp.dot(h, Wq[...], preferred_element_type=jnp.float32) + bq[...]
    k_ref[...] = jnp.dot(h, Wk[...], preferred_element_type=jnp.float32) + bk[...]
    v_ref[...] = jnp.dot(h, Wv[...], preferred_element_type=jnp.float32) + bv[...]
    s_ref[...] = jnp.dot(h, Ws[...], preferred_element_type=jnp.float32) + bs[...]


def _qkvs(h, lp):
    return pl.pallas_call(
        _qkvs_body,
        grid=(N // BN,),
        in_specs=[pl.BlockSpec((BN, HID), lambda i: (i, 0))] +
                 [_full((HID, HID)), _full((1, HID))] * 4,
        out_specs=[pl.BlockSpec((BN, HID), lambda i: (i, 0))] * 4,
        out_shape=[jax.ShapeDtypeStruct((N, HID), jnp.float32)] * 4,
    )(h, lp["Wq"], lp["bq"].reshape(1, HID), lp["Wk"], lp["bk"].reshape(1, HID),
      lp["Wv"], lp["bv"].reshape(1, HID), lp["Ws"], lp["bs"].reshape(1, HID))


def _et_body(ea_ref, We, o_ref):
    o_ref[...] = jnp.dot(ea_ref[...], We[...], preferred_element_type=jnp.float32)


def _edge_transform(ea, We):
    return pl.pallas_call(
        _et_body,
        grid=(E // BE,),
        in_specs=[pl.BlockSpec((BE, HID), lambda i: (i, 0)), _full((HID, HID))],
        out_specs=pl.BlockSpec((BE, HID), lambda i: (i, 0)),
        out_shape=jax.ShapeDtypeStruct((E, HID), jnp.float32),
    )(ea, We)


def _post_body(agg_ref, hs_ref, h_ref, g_ref, b_ref, o_ref):
    out = agg_ref[...] + hs_ref[...]
    out = jnp.maximum(_ln(out, g_ref[...], b_ref[...]), 0.0)
    o_ref[...] = out + h_ref[...]


def _post(agg, hs, h, lp):
    return pl.pallas_call(
        _post_body,
        grid=(N // BN,),
        in_specs=[pl.BlockSpec((BN, HID), lambda i: (i, 0))] * 3 +
                 [_full((1, HID)), _full((1, HID))],
        out_specs=pl.BlockSpec((BN, HID), lambda i: (i, 0)),
        out_shape=jax.ShapeDtypeStruct((N, HID), jnp.float32),
    )(agg, hs, h, lp["ln_g"].reshape(1, HID), lp["ln_b"].reshape(1, HID))


def _pool_body(h_ref, W1, b1, W2, b2, o_ref):
    t = jnp.tanh(jnp.dot(h_ref[...], W1[...], preferred_element_type=jnp.float32) + b1[...])
    lg = jnp.dot(t, W2[...], preferred_element_type=jnp.float32) + b2[...]
    o_ref[...] = jnp.broadcast_to(lg, (BN, 8))


def _pool_logits(h, W1, b1, W2, b2):
    out = pl.pallas_call(
        _pool_body,
        grid=(N // BN,),
        in_specs=[pl.BlockSpec((BN, HID), lambda i: (i, 0)),
                  _full((HID, HID)), _full((1, HID)),
                  _full((HID, 1)), _full((1, 1))],
        out_specs=pl.BlockSpec((BN, 8), lambda i: (i, 0)),
        out_shape=jax.ShapeDtypeStruct((N, 8), jnp.float32),
    )(h, W1, b1.reshape(1, HID), W2, b2.reshape(1, 1))
    return out[:, 0]


def _proj_body(g_ref, W1, b1, g1, be1, W2, b2, g2, be2, W3, b3, o_ref):
    g = g_ref[...]
    g = jnp.maximum(_ln(jnp.dot(g, W1[...], preferred_element_type=jnp.float32) + b1[...],
                        g1[...], be1[...]), 0.0)
    g = jnp.maximum(_ln(jnp.dot(g, W2[...], preferred_element_type=jnp.float32) + b2[...],
                        g2[...], be2[...]), 0.0)
    g = jnp.dot(g, W3[...], preferred_element_type=jnp.float32) + b3[...]
    nrm = jnp.sqrt(jnp.sum(g * g, axis=-1, keepdims=True))
    o_ref[...] = g / jnp.maximum(nrm, 1e-12)


def _proj_head(g, p):
    H2 = HID * 2
    return pl.pallas_call(
        _proj_body,
        grid=(1,),
        in_specs=[_full((NG, HID)),
                  _full((HID, H2)), _full((1, H2)), _full((1, H2)), _full((1, H2)),
                  _full((H2, HID)), _full((1, HID)), _full((1, HID)), _full((1, HID)),
                  _full((HID, 128)), _full((1, 128))],
        out_specs=_full((NG, 128)),
        out_shape=jax.ShapeDtypeStruct((NG, 128), jnp.float32),
    )(g, p["proj_W1"], p["proj_b1"].reshape(1, H2), p["proj_g1"].reshape(1, H2),
      p["proj_be1"].reshape(1, H2), p["proj_W2"], p["proj_b2"].reshape(1, HID),
      p["proj_g2"].reshape(1, HID), p["proj_be2"].reshape(1, HID),
      p["proj_W3"], p["proj_b3"].reshape(1, 128))


def _two_row_collapse(tables, fw):
    sig = jax.nn.sigmoid(fw)
    base = sum(sig[i] * tables[i][0] for i in range(len(tables)))
    D = jnp.stack([sig[i] * (tables[i][1] - tables[i][0]) for i in range(len(tables))], 0)
    return base.reshape(1, HID), D


def kernel(params, x, edge_index, edge_attr, batch):
    p = params
    xf = x.astype(jnp.float32)
    ef = edge_attr.astype(jnp.float32)
    src, dst = edge_index[0], edge_index[1]

    base_a, D_a = _two_row_collapse(p["atom_emb"], p["atom_fw"])
    base_b, D_b = _two_row_collapse(p["bond_emb"], p["bond_fw"])
    pos0 = p["pos"][0].reshape(1, HID)
    posd = (p["pos"][1] - p["pos"][0]).reshape(1, HID)

    h = _atom_encode(xf, D_a, base_a, p["atom_W"], p["atom_b"].reshape(1, HID),
                     p["atom_g"].reshape(1, HID), p["atom_be"].reshape(1, HID),
                     pos0, posd)
    ea = _bond_encode(ef, D_b, base_b, p["bond_W"], p["bond_b"].reshape(1, HID),
                      p["bond_g"].reshape(1, HID), p["bond_be"].reshape(1, HID))

    for lp in p["layers"]:
        q, k, v, hs = _qkvs(h, lp)
        e = _edge_transform(ea, lp["We"])

        qh = q.reshape(N, HEADS, CH)
        kh = k.reshape(N, HEADS, CH)
        vh = v.reshape(N, HEADS, CH)
        eh = e.reshape(E, HEADS, CH)
        ke = kh[src] + eh
        logit = jnp.sum(qh[dst] * ke, axis=-1) * 0.25
        m = jax.ops.segment_max(logit, dst, num_segments=N)
        ex = jnp.exp(logit - m[dst])
        s = jax.ops.segment_sum(ex, dst, num_segments=N)
        alpha = ex / (s[dst] + 1e-16)
        msg = (vh[src] + eh) * alpha[..., None]
        agg = jax.ops.segment_sum(msg, dst, num_segments=N).reshape(N, HID)
        h = _post(agg, hs, h, lp)

    logits = _pool_logits(h, p["pool_W1"], p["pool_b1"], p["pool_W2"], p["pool_b2"])
    m = jax.ops.segment_max(logits, batch, num_segments=NG, indices_are_sorted=True)
    ex = jnp.exp(logits - m[batch])
    s = jax.ops.segment_sum(ex, batch, num_segments=NG, indices_are_sorted=True)
    w = ex / (s[batch] + 1e-16)
    g = jax.ops.segment_sum(h * w[:, None], batch, num_segments=NG,
                            indices_are_sorted=True)
    return _proj_head(g, p)
